# baseline (speedup 1.0000x reference)
import functools

import jax
import jax.numpy as jnp
from jax import lax
from jax.experimental import pallas as pl
from jax.experimental.pallas import tpu as pltpu
from jax.experimental.pallas import tpu_sc as plsc

N = 10000
E = 320000
D = 128

NC = 2
NS = 16
NW = NC * NS
EPW = E // NW
K = 80
CHUNKS = EPW // K
NB = 3
RPT = 624
RREM = N - RPT * NS
LANES = 16
DV = D // LANES


def _sc_body(x_hbm, col_hbm, row_hbm, w_hbm, zeros_hbm, out_hbm,
             colstage, rowbufs, wbufs, gbuf0, gbuf1, gbuf2, acc,
             rwsem0, rwsem1, rwsem2, gsem0, gsem1, gsem2):
    c = lax.axis_index("c")
    s = lax.axis_index("s")
    wid = c * NS + s

    pltpu.sync_copy(col_hbm.at[wid], colstage)

    pltpu.sync_copy(zeros_hbm.at[pl.ds(s * RPT, RPT)],
                    acc.at[pl.ds(s * RPT, RPT)])

    @pl.when(s == NS - 1)
    def _():
        pltpu.sync_copy(zeros_hbm.at[pl.ds(RPT * NS, RREM)],
                        acc.at[pl.ds(RPT * NS, RREM)])

    plsc.subcore_barrier()

    gbufs = (gbuf0, gbuf1, gbuf2)
    gsems = (gsem0, gsem1, gsem2)
    rwsems = (rwsem0, rwsem1, rwsem2)

    def stage_rw(t, slot):
        pltpu.async_copy(row_hbm.at[wid, t], rowbufs.at[slot], rwsems[slot])
        pltpu.async_copy(w_hbm.at[wid, t], wbufs.at[slot], rwsems[slot])

    def drain_rw(slot):
        pltpu.make_async_copy(row_hbm.at[0, 0], rowbufs.at[slot],
                              rwsems[slot]).wait()
        pltpu.make_async_copy(w_hbm.at[0, 0], wbufs.at[slot],
                              rwsems[slot]).wait()

    def issue_gather(t, slot):
        pltpu.async_copy(x_hbm.at[colstage.at[pl.ds(t * K, K)]],
                         gbufs[slot], gsems[slot])

    for t in range(NB):
        stage_rw(t, t)
    for t in range(2):
        issue_gather(t, t)

    def chunk_body(j, carry):
        for p in range(NB):
            @pl.when(j % NB == p)
            def _():
                p2 = (p + 2) % NB
                @pl.when(j + 2 < CHUNKS)
                def _():
                    issue_gather(j + 2, p2)

                gbuf = gbufs[p]
                pltpu.make_async_copy(x_hbm.at[pl.ds(0, K)], gbuf,
                                      gsems[p]).wait()
                drain_rw(p)

                def grp_body(g, carry2):
                    wg = wbufs[p, pl.ds(g * LANES, LANES)]
                    for i in range(LANES):
                        e = g * LANES + i
                        wsplat = jnp.full((LANES,), wg[i], jnp.float32)
                        for d in range(DV):
                            sl = pl.ds(d * LANES, LANES)
                            gbuf[e, sl] = gbuf[e, sl] * wsplat
                    return carry2

                lax.fori_loop(0, K // LANES, grp_body, 0)

                pltpu.sync_copy(gbuf, acc.at[rowbufs.at[p]], add=True)

                @pl.when(j + NB < CHUNKS)
                def _():
                    stage_rw(j + NB, p)
        return carry

    lax.fori_loop(0, CHUNKS, chunk_body, 0)
    plsc.subcore_barrier()

    pltpu.sync_copy(acc.at[pl.ds(s * RPT, RPT)],
                    out_hbm.at[c, pl.ds(s * RPT, RPT)])

    @pl.when(s == NS - 1)
    def _():
        pltpu.sync_copy(acc.at[pl.ds(RPT * NS, RREM)],
                        out_hbm.at[c, pl.ds(RPT * NS, RREM)])


@jax.jit
def _sc_spmm(x, col2, row3, w3, zeros):
    mesh = plsc.VectorSubcoreMesh(core_axis_name="c", subcore_axis_name="s")
    f = pl.kernel(
        _sc_body,
        out_type=jax.ShapeDtypeStruct((NC, N, D), jnp.float32),
        mesh=mesh,
        scratch_types=[
            pltpu.VMEM((EPW,), jnp.int32),
            pltpu.VMEM((NB, K), jnp.int32),
            pltpu.VMEM((NB, K), jnp.float32),
            pltpu.VMEM((K, D), jnp.float32),
            pltpu.VMEM((K, D), jnp.float32),
            pltpu.VMEM((K, D), jnp.float32),
            pltpu.VMEM_SHARED((N, D), jnp.float32),
            pltpu.SemaphoreType.DMA,
            pltpu.SemaphoreType.DMA,
            pltpu.SemaphoreType.DMA,
            pltpu.SemaphoreType.DMA,
            pltpu.SemaphoreType.DMA,
            pltpu.SemaphoreType.DMA,
        ],
    )
    return f(x, col2, row3, w3, zeros)


def _add_body(a_ref, b_ref, o_ref):
    o_ref[...] = a_ref[...] + b_ref[...]


@jax.jit
def _tc_combine(partials):
    blk = 1000
    return pl.pallas_call(
        _add_body,
        out_shape=jax.ShapeDtypeStruct((N, D), jnp.float32),
        grid=(N // blk,),
        in_specs=[
            pl.BlockSpec((blk, D), lambda i: (i, 0)),
            pl.BlockSpec((blk, D), lambda i: (i, 0)),
        ],
        out_specs=pl.BlockSpec((blk, D), lambda i: (i, 0)),
    )(partials[0], partials[1])


def kernel(x, edge_index, edge_weight):
    zeros = jnp.zeros((N, D), jnp.float32)
    col2 = edge_index[1].reshape(NW, EPW)
    row3 = edge_index[0].reshape(NW, CHUNKS, K)
    w3 = edge_weight.reshape(NW, CHUNKS, K)
    partials = _sc_spmm(x, col2, row3, w3, zeros)
    return _tc_combine(partials)


# speedup vs baseline: 1.0307x; 1.0307x over previous
import functools

import jax
import jax.numpy as jnp
from jax import lax
from jax.experimental import pallas as pl
from jax.experimental.pallas import tpu as pltpu
from jax.experimental.pallas import tpu_sc as plsc

N = 10000
E = 320000
D = 128

NC = 2
NS = 16
NW = NC * NS
EPW = E // NW
K = 80
CHUNKS = EPW // K
NB = 3
RPT = 624
RREM = N - RPT * NS
LANES = 16
DV = D // LANES


def _sc_body(x_hbm, col_hbm, row_hbm, w_hbm, out_hbm,
             colstage, rowbufs, wbufs, gbuf0, gbuf1, gbuf2, acc,
             rwsem0, rwsem1, rwsem2, gsem0, gsem1, gsem2):
    c = lax.axis_index("c")
    s = lax.axis_index("s")
    wid = c * NS + s

    def zrow_body(r, carry):
        for d in range(DV):
            gbuf0[r, pl.ds(d * LANES, LANES)] = jnp.zeros((LANES,),
                                                          jnp.float32)
        return carry

    lax.fori_loop(0, K, zrow_body, 0)
    for t in range(RPT // K):
        pltpu.sync_copy(gbuf0, acc.at[pl.ds(s * RPT + t * K, K)])
    pltpu.sync_copy(gbuf0.at[pl.ds(0, RPT - (RPT // K) * K)],
                    acc.at[pl.ds(s * RPT + (RPT // K) * K,
                                 RPT - (RPT // K) * K)])

    @pl.when(s == NS - 1)
    def _():
        pltpu.sync_copy(gbuf0.at[pl.ds(0, RREM)],
                        acc.at[pl.ds(RPT * NS, RREM)])

    pltpu.sync_copy(col_hbm.at[wid], colstage)

    plsc.subcore_barrier()

    gbufs = (gbuf0, gbuf1, gbuf2)
    gsems = (gsem0, gsem1, gsem2)
    rwsems = (rwsem0, rwsem1, rwsem2)

    def stage_rw(t, slot):
        pltpu.async_copy(row_hbm.at[wid, t], rowbufs.at[slot], rwsems[slot])
        pltpu.async_copy(w_hbm.at[wid, t], wbufs.at[slot], rwsems[slot])

    def drain_rw(slot):
        pltpu.make_async_copy(row_hbm.at[0, 0], rowbufs.at[slot],
                              rwsems[slot]).wait()
        pltpu.make_async_copy(w_hbm.at[0, 0], wbufs.at[slot],
                              rwsems[slot]).wait()

    def issue_gather(t, slot):
        pltpu.async_copy(x_hbm.at[colstage.at[pl.ds(t * K, K)]],
                         gbufs[slot], gsems[slot])

    for t in range(NB):
        stage_rw(t, t)
    for t in range(2):
        issue_gather(t, t)

    def chunk_body(j, carry):
        for p in range(NB):
            @pl.when(j % NB == p)
            def _():
                p2 = (p + 2) % NB
                @pl.when(j + 2 < CHUNKS)
                def _():
                    issue_gather(j + 2, p2)

                gbuf = gbufs[p]
                pltpu.make_async_copy(x_hbm.at[pl.ds(0, K)], gbuf,
                                      gsems[p]).wait()
                drain_rw(p)

                def grp_body(g, carry2):
                    wg = wbufs[p, pl.ds(g * LANES, LANES)]
                    for i in range(LANES):
                        e = g * LANES + i
                        wsplat = jnp.full((LANES,), wg[i], jnp.float32)
                        for d in range(DV):
                            sl = pl.ds(d * LANES, LANES)
                            gbuf[e, sl] = gbuf[e, sl] * wsplat
                    return carry2

                lax.fori_loop(0, K // LANES, grp_body, 0)

                pltpu.sync_copy(gbuf, acc.at[rowbufs.at[p]], add=True)

                @pl.when(j + NB < CHUNKS)
                def _():
                    stage_rw(j + NB, p)
        return carry

    lax.fori_loop(0, CHUNKS, chunk_body, 0)
    plsc.subcore_barrier()

    pltpu.sync_copy(acc.at[pl.ds(s * RPT, RPT)],
                    out_hbm.at[c, pl.ds(s * RPT, RPT)])

    @pl.when(s == NS - 1)
    def _():
        pltpu.sync_copy(acc.at[pl.ds(RPT * NS, RREM)],
                        out_hbm.at[c, pl.ds(RPT * NS, RREM)])


@jax.jit
def _sc_spmm(x, col2, row3, w3):
    mesh = plsc.VectorSubcoreMesh(core_axis_name="c", subcore_axis_name="s")
    f = pl.kernel(
        _sc_body,
        out_type=jax.ShapeDtypeStruct((NC, N, D), jnp.float32),
        mesh=mesh,
        scratch_types=[
            pltpu.VMEM((EPW,), jnp.int32),
            pltpu.VMEM((NB, K), jnp.int32),
            pltpu.VMEM((NB, K), jnp.float32),
            pltpu.VMEM((K, D), jnp.float32),
            pltpu.VMEM((K, D), jnp.float32),
            pltpu.VMEM((K, D), jnp.float32),
            pltpu.VMEM_SHARED((N, D), jnp.float32),
            pltpu.SemaphoreType.DMA,
            pltpu.SemaphoreType.DMA,
            pltpu.SemaphoreType.DMA,
            pltpu.SemaphoreType.DMA,
            pltpu.SemaphoreType.DMA,
            pltpu.SemaphoreType.DMA,
        ],
    )
    return f(x, col2, row3, w3)


def _add_body(a_ref, b_ref, o_ref):
    o_ref[...] = a_ref[...] + b_ref[...]


@jax.jit
def _tc_combine(partials):
    blk = 1000
    return pl.pallas_call(
        _add_body,
        out_shape=jax.ShapeDtypeStruct((N, D), jnp.float32),
        grid=(N // blk,),
        in_specs=[
            pl.BlockSpec((blk, D), lambda i: (i, 0)),
            pl.BlockSpec((blk, D), lambda i: (i, 0)),
        ],
        out_specs=pl.BlockSpec((blk, D), lambda i: (i, 0)),
    )(partials[0], partials[1])


def kernel(x, edge_index, edge_weight):
    col2 = edge_index[1].reshape(NW, EPW)
    row3 = edge_index[0].reshape(NW, CHUNKS, K)
    w3 = edge_weight.reshape(NW, CHUNKS, K)
    partials = _sc_spmm(x, col2, row3, w3)
    return _tc_combine(partials)


# speedup vs baseline: 1.0762x; 1.0441x over previous
import jax
import jax.numpy as jnp
from jax import lax
from jax.experimental import pallas as pl
from jax.experimental.pallas import tpu as pltpu
from jax.experimental.pallas import tpu_sc as plsc

N = 10000
E = 320000
D = 128

NC = 2
NS = 16
NW = NC * NS
EPW = E // NW
K = 80
CHUNKS = EPW // K
NB = 4
RPT = 624
RREM = N - RPT * NS
LANES = 16
DV = D // LANES


def _sc_body(x_hbm, col_hbm, row_hbm, w_hbm, out_hbm,
             colbufs, rowbufs, wbufs, gbuf0, gbuf1, gbuf2, gbuf3, acc,
             csem0, csem1, csem2, csem3,
             rwsem0, rwsem1, rwsem2, rwsem3,
             gsem0, gsem1, gsem2, gsem3,
             ssem0, ssem1, ssem2, ssem3):
    c = lax.axis_index("c")
    s = lax.axis_index("s")
    wid = c * NS + s

    def zrow_body(r, carry):
        for d in range(DV):
            gbuf0[r, pl.ds(d * LANES, LANES)] = jnp.zeros((LANES,),
                                                          jnp.float32)
        return carry

    lax.fori_loop(0, K, zrow_body, 0)
    for t in range(RPT // K):
        pltpu.sync_copy(gbuf0, acc.at[pl.ds(s * RPT + t * K, K)])
    pltpu.sync_copy(gbuf0.at[pl.ds(0, RPT - (RPT // K) * K)],
                    acc.at[pl.ds(s * RPT + (RPT // K) * K,
                                 RPT - (RPT // K) * K)])

    @pl.when(s == NS - 1)
    def _():
        pltpu.sync_copy(gbuf0.at[pl.ds(0, RREM)],
                        acc.at[pl.ds(RPT * NS, RREM)])

    plsc.subcore_barrier()

    gbufs = (gbuf0, gbuf1, gbuf2, gbuf3)
    gsems = (gsem0, gsem1, gsem2, gsem3)
    csems = (csem0, csem1, csem2, csem3)
    rwsems = (rwsem0, rwsem1, rwsem2, rwsem3)
    ssems = (ssem0, ssem1, ssem2, ssem3)

    def stage_col(t, slot):
        pltpu.async_copy(col_hbm.at[wid, t], colbufs.at[slot], csems[slot])

    def drain_col(slot):
        pltpu.make_async_copy(col_hbm.at[0, 0], colbufs.at[slot],
                              csems[slot]).wait()

    def stage_rw(t, slot):
        pltpu.async_copy(row_hbm.at[wid, t], rowbufs.at[slot], rwsems[slot])
        pltpu.async_copy(w_hbm.at[wid, t], wbufs.at[slot], rwsems[slot])

    def drain_rw(slot):
        pltpu.make_async_copy(row_hbm.at[0, 0], rowbufs.at[slot],
                              rwsems[slot]).wait()
        pltpu.make_async_copy(w_hbm.at[0, 0], wbufs.at[slot],
                              rwsems[slot]).wait()

    def issue_gather(slot):
        pltpu.async_copy(x_hbm.at[colbufs.at[slot]], gbufs[slot],
                         gsems[slot])

    def drain_scatter(slot):
        pltpu.make_async_copy(x_hbm.at[pl.ds(0, K)], gbufs[slot],
                              ssems[slot]).wait()

    for t in range(NB):
        stage_col(t, t)
    for t in range(2):
        stage_rw(t, t)
    for t in range(2):
        drain_col(t)
        issue_gather(t)

    def chunk_body(j, carry):
        for p in range(NB):
            @pl.when(j % NB == p)
            def _():
                q = (p + 2) % NB
                @pl.when(j + 2 < CHUNKS)
                def _():
                    @pl.when(j >= 2)
                    def _():
                        drain_scatter(q)
                    drain_col(q)
                    issue_gather(q)
                    stage_rw(j + 2, q)

                gbuf = gbufs[p]
                pltpu.make_async_copy(x_hbm.at[pl.ds(0, K)], gbuf,
                                      gsems[p]).wait()
                drain_rw(p)

                @pl.when(j + NB < CHUNKS)
                def _():
                    stage_col(j + NB, p)

                def grp_body(g, carry2):
                    wg = wbufs[p, pl.ds(g * LANES, LANES)]
                    for i in range(LANES):
                        e = g * LANES + i
                        wsplat = jnp.full((LANES,), wg[i], jnp.float32)
                        for d in range(DV):
                            sl = pl.ds(d * LANES, LANES)
                            gbuf[e, sl] = gbuf[e, sl] * wsplat
                    return carry2

                lax.fori_loop(0, K // LANES, grp_body, 0)

                pltpu.async_copy(gbuf, acc.at[rowbufs.at[p]], ssems[p],
                                 add=True)
        return carry

    lax.fori_loop(0, CHUNKS, chunk_body, 0)

    for slot in range(NB):
        drain_scatter(slot)
    plsc.subcore_barrier()

    pltpu.sync_copy(acc.at[pl.ds(s * RPT, RPT)],
                    out_hbm.at[c, pl.ds(s * RPT, RPT)])

    @pl.when(s == NS - 1)
    def _():
        pltpu.sync_copy(acc.at[pl.ds(RPT * NS, RREM)],
                        out_hbm.at[c, pl.ds(RPT * NS, RREM)])


@jax.jit
def _sc_spmm(x, col3, row3, w3):
    mesh = plsc.VectorSubcoreMesh(core_axis_name="c", subcore_axis_name="s")
    f = pl.kernel(
        _sc_body,
        out_type=jax.ShapeDtypeStruct((NC, N, D), jnp.float32),
        mesh=mesh,
        scratch_types=[
            pltpu.VMEM((NB, K), jnp.int32),
            pltpu.VMEM((NB, K), jnp.int32),
            pltpu.VMEM((NB, K), jnp.float32),
            pltpu.VMEM((K, D), jnp.float32),
            pltpu.VMEM((K, D), jnp.float32),
            pltpu.VMEM((K, D), jnp.float32),
            pltpu.VMEM((K, D), jnp.float32),
            pltpu.VMEM_SHARED((N, D), jnp.float32),
            pltpu.SemaphoreType.DMA,
            pltpu.SemaphoreType.DMA,
            pltpu.SemaphoreType.DMA,
            pltpu.SemaphoreType.DMA,
            pltpu.SemaphoreType.DMA,
            pltpu.SemaphoreType.DMA,
            pltpu.SemaphoreType.DMA,
            pltpu.SemaphoreType.DMA,
            pltpu.SemaphoreType.DMA,
            pltpu.SemaphoreType.DMA,
            pltpu.SemaphoreType.DMA,
            pltpu.SemaphoreType.DMA,
            pltpu.SemaphoreType.DMA,
            pltpu.SemaphoreType.DMA,
            pltpu.SemaphoreType.DMA,
            pltpu.SemaphoreType.DMA,
        ],
    )
    return f(x, col3, row3, w3)


def _add_body(a_ref, b_ref, o_ref):
    o_ref[...] = a_ref[...] + b_ref[...]


@jax.jit
def _tc_combine(partials):
    blk = 1000
    return pl.pallas_call(
        _add_body,
        out_shape=jax.ShapeDtypeStruct((N, D), jnp.float32),
        grid=(N // blk,),
        in_specs=[
            pl.BlockSpec((blk, D), lambda i: (i, 0)),
            pl.BlockSpec((blk, D), lambda i: (i, 0)),
        ],
        out_specs=pl.BlockSpec((blk, D), lambda i: (i, 0)),
    )(partials[0], partials[1])


def kernel(x, edge_index, edge_weight):
    col3 = edge_index[1].reshape(NW, CHUNKS, K)
    row3 = edge_index[0].reshape(NW, CHUNKS, K)
    w3 = edge_weight.reshape(NW, CHUNKS, K)
    partials = _sc_spmm(x, col3, row3, w3)
    return _tc_combine(partials)
